# baseline (speedup 1.0000x reference)
import jax
import jax.numpy as jnp
from jax import lax
from jax.experimental import pallas as pl
from jax.experimental.pallas import tpu as pltpu
from jax.experimental.pallas import tpu_sc as plsc

D = 256
HALF = 128
N_NODES = 10000
N_SC = 2
N_TILE = 16
LANES = 16
CHUNK = 128
EC = 80
EPT = EC * CHUNK
E_PAD = N_TILE * EPT
ND_PAD = 10240
RPT = ND_PAD // N_TILE


DEGW = 16


NBUF = 2


HROWS = ND_PAD // LANES


SGC = 8


def _sc_body(gidx_hbm, col_hbm, cflat_hbm, xr_hbm, t_out, parts_out,
             ibuf, cbuf, cflat, gbuf, hist, acc,
             sg0, sg1):
    c = lax.axis_index("c")
    s = lax.axis_index("s")
    w = c * N_TILE + s
    zero16 = jnp.zeros((LANES,), jnp.float32)
    ones16 = jnp.ones((LANES,), jnp.float32)
    sgs = (sg0, sg1)

    @pl.loop(0, CHUNK)
    def _(i):
        for j in range(HALF // LANES):
            gbuf[0, i, pl.ds(j * LANES, LANES)] = zero16
    for k in range(RPT // CHUNK):
        pltpu.sync_copy(gbuf.at[0], acc.at[pl.ds(s * RPT + k * CHUNK, CHUNK)])

    izero16 = jnp.zeros((LANES,), jnp.int32)

    @pl.loop(0, ND_PAD // LANES)
    def _(i):
        hist[pl.ds(i * LANES, LANES)] = izero16

    plsc.subcore_barrier()

    iota16 = lax.iota(jnp.int32, LANES)

    @pl.loop(0, EC // SGC)
    def _(sg):
        base = sg * SGC
        pltpu.sync_copy(gidx_hbm.at[pl.ds(w * EC + base, SGC)], ibuf)
        pltpu.sync_copy(col_hbm.at[pl.ds(s * EC + base, SGC)], cbuf)
        pltpu.sync_copy(cflat_hbm.at[pl.ds(s * EPT + base * CHUNK, SGC * CHUNK)],
                        cflat)

        def _mk(k):
            return pltpu.make_async_copy(xr_hbm.at[ibuf.at[k]],
                                         gbuf.at[k % NBUF], sgs[k % NBUF])

        descs = {k: _mk(k) for k in range(NBUF)}
        for k in range(NBUF):
            descs[k].start()

        @pl.loop(0, SGC * CHUNK // LANES)
        def _(q):
            cv = cflat[pl.ds(q * LANES, LANES)]
            for l in range(LANES):
                v = cv[l]
                b0 = (v >> 4) << 4
                win = hist[pl.ds(b0, LANES)]
                hist[pl.ds(b0, LANES)] = win + jnp.where(
                    iota16 == (v - b0), 1, 0).astype(jnp.int32)

        for k in range(SGC):
            descs[k].wait()
            pltpu.sync_copy(gbuf.at[k % NBUF], acc.at[cbuf.at[k]], add=True)
            if k + NBUF < SGC:
                descs[k + NBUF] = _mk(k + NBUF)
                descs[k + NBUF].start()

    plsc.subcore_barrier()

    pltpu.sync_copy(acc.at[pl.ds(s * RPT, RPT)],
                    t_out.at[pl.ds(c * ND_PAD + s * RPT, RPT)])
    pltpu.sync_copy(hist, parts_out.at[pl.ds(w * ND_PAD, ND_PAD)])


_sc_call = pl.kernel(
    _sc_body,
    out_type=[
        jax.ShapeDtypeStruct((N_SC * ND_PAD, HALF), jnp.float32),
        jax.ShapeDtypeStruct((N_SC * N_TILE * ND_PAD,), jnp.int32),
    ],
    mesh=plsc.VectorSubcoreMesh(core_axis_name="c", subcore_axis_name="s"),
    scratch_types=[
        pltpu.VMEM((SGC, CHUNK), jnp.int32),
        pltpu.VMEM((SGC, CHUNK), jnp.int32),
        pltpu.VMEM((SGC * CHUNK,), jnp.int32),
        pltpu.VMEM((NBUF, CHUNK, HALF), jnp.float32),
        pltpu.VMEM((ND_PAD,), jnp.int32),
        pltpu.VMEM_SHARED((ND_PAD, HALF), jnp.float32),
        pltpu.SemaphoreType.DMA,
        pltpu.SemaphoreType.DMA,
    ],
)


BLK = 256


def _tc_body(h_ref, tl_ref, tr_ref, parts_ref, ws_t_ref, b_ref, w_t_ref,
             rel_ref, wrel_t_ref, o_ref):
    deg = jnp.sum(parts_ref[...], axis=0).astype(jnp.float32).reshape(BLK, 1)
    pos = deg > 0.0
    dinv = jnp.where(pos, 1.0 / deg, 0.0)
    hr = jnp.dot(rel_ref[...], wrel_t_ref[...],
                 preferred_element_type=jnp.float32)
    sl = tl_ref[...] * dinv - jnp.where(pos, hr[:, :HALF], 0.0)
    sr = tr_ref[...] * dinv - jnp.where(pos, hr[:, HALF:], 0.0)
    a = jnp.dot(h_ref[...], ws_t_ref[...], preferred_element_type=jnp.float32)
    a = a + jnp.dot(sl, w_t_ref[:HALF, :], preferred_element_type=jnp.float32)
    a = a + jnp.dot(sr, w_t_ref[HALF:, :], preferred_element_type=jnp.float32)
    a = a + b_ref[...]
    o_ref[...] = jnp.where(a > 0.0, a, jnp.exp(a) - 1.0)


_tc_call = pl.pallas_call(
    _tc_body,
    grid=(ND_PAD // BLK,),
    in_specs=[
        pl.BlockSpec((BLK, D), lambda i: (i, 0)),
        pl.BlockSpec((BLK, HALF), lambda i: (i, 0)),
        pl.BlockSpec((BLK, HALF), lambda i: (i, 0)),
        pl.BlockSpec((N_TILE, BLK), lambda i: (0, i)),
        pl.BlockSpec((D, D), lambda i: (0, 0)),
        pl.BlockSpec((1, D), lambda i: (0, 0)),
        pl.BlockSpec((D, D), lambda i: (0, 0)),
        pl.BlockSpec((1, D), lambda i: (0, 0)),
        pl.BlockSpec((D, D), lambda i: (0, 0)),
    ],
    out_specs=pl.BlockSpec((BLK, D), lambda i: (i, 0)),
    out_shape=jax.ShapeDtypeStruct((ND_PAD, D), jnp.float32),
)


def _prep_edges(row, col):
    pad = E_PAD - row.shape[0]
    rowp = jnp.concatenate([row, jnp.zeros((pad,), jnp.int32)])
    colp = jnp.concatenate([col, jnp.full((pad,), N_NODES, jnp.int32)])
    rowp = rowp.reshape(N_TILE, EC, CHUNK)
    gidx = jnp.stack([rowp * 2, rowp * 2 + 1]).reshape(N_SC * N_TILE * EC, CHUNK)
    return gidx, colp.reshape(N_TILE * EC, CHUNK), colp


def kernel(h_project, h_company, edge_index_fwd, edge_index_bwd,
           W_self_p, b_self_p, W_self_c, b_self_c,
           W_out, W_in, W_rel, rel_fwd, rel_bwd):
    ei_f = edge_index_fwd.astype(jnp.int32)
    ei_b = edge_index_bwd.astype(jnp.int32)
    gidxf, colf_t, colf_flat = _prep_edges(ei_f[0], ei_f[1])
    gidxb, colb_t, colb_flat = _prep_edges(ei_b[0], ei_b[1])
    xr_p = h_project.reshape(N_NODES * 2, HALF)
    xr_c = h_company.reshape(N_NODES * 2, HALF)
    t_c, parts_c = _sc_call(gidxf, colf_t, colf_flat, xr_p)
    xr_c, t_c = lax.optimization_barrier((xr_c, t_c))
    t_p, parts_p = _sc_call(gidxb, colb_t, colb_flat, xr_c)

    hp_pad = jnp.pad(h_project, ((0, ND_PAD - N_NODES), (0, 0)))
    hc_pad = jnp.pad(h_company, ((0, ND_PAD - N_NODES), (0, 0)))
    wrel_t = W_rel.T

    parts_p = parts_p.reshape(N_SC, N_TILE, ND_PAD)[0]
    parts_c = parts_c.reshape(N_SC, N_TILE, ND_PAD)[0]
    out_p = _tc_call(hp_pad, t_p[:ND_PAD], t_p[ND_PAD:], parts_p,
                     W_self_p.T, b_self_p.reshape(1, D), W_in.T,
                     rel_bwd.reshape(1, D), wrel_t)
    out_c = _tc_call(hc_pad, t_c[:ND_PAD], t_c[ND_PAD:], parts_c,
                     W_self_c.T, b_self_c.reshape(1, D), W_out.T,
                     rel_fwd.reshape(1, D), wrel_t)
    return out_p[:N_NODES], out_c[:N_NODES]


# speedup vs baseline: 1.0054x; 1.0054x over previous
import jax
import jax.numpy as jnp
from jax import lax
from jax.experimental import pallas as pl
from jax.experimental.pallas import tpu as pltpu
from jax.experimental.pallas import tpu_sc as plsc

D = 256
HALF = 128
N_NODES = 10000
N_SC = 2
N_TILE = 16
LANES = 16
CHUNK = 128
EC = 80
EPT = EC * CHUNK
E_PAD = N_TILE * EPT
ND_PAD = 10240
RPT = ND_PAD // N_TILE


NBUF = 2
SGC = 8


def _sc_body(gidx_hbm, col_hbm, cflat_hbm, xr_hbm, t_out, parts_out,
             ibuf, cbuf, cflat, gbuf, hist, acc,
             sg0, sg1):
    c = lax.axis_index("c")
    s = lax.axis_index("s")
    w = c * N_TILE + s
    zero16 = jnp.zeros((LANES,), jnp.float32)
    sgs = (sg0, sg1)

    @pl.loop(0, CHUNK)
    def _(i):
        for j in range(HALF // LANES):
            gbuf[0, i, pl.ds(j * LANES, LANES)] = zero16
    for k in range(RPT // CHUNK):
        pltpu.sync_copy(gbuf.at[0], acc.at[pl.ds(s * RPT + k * CHUNK, CHUNK)])

    izero16 = jnp.zeros((LANES,), jnp.int32)

    @pl.loop(0, ND_PAD // LANES)
    def _(i):
        hist[pl.ds(i * LANES, LANES)] = izero16

    plsc.subcore_barrier()

    iota16 = lax.iota(jnp.int32, LANES)

    @pl.loop(0, EC // SGC)
    def _(sg):
        base = sg * SGC
        pltpu.sync_copy(gidx_hbm.at[pl.ds(w * EC + base, SGC)], ibuf)
        pltpu.sync_copy(col_hbm.at[pl.ds(s * EC + base, SGC)], cbuf)
        pltpu.sync_copy(cflat_hbm.at[pl.ds(s * EPT + base * CHUNK, SGC * CHUNK)],
                        cflat)

        def _mk(k):
            return pltpu.make_async_copy(xr_hbm.at[ibuf.at[k]],
                                         gbuf.at[k % NBUF], sgs[k % NBUF])

        descs = {k: _mk(k) for k in range(NBUF)}
        for k in range(NBUF):
            descs[k].start()

        @pl.loop(0, SGC * CHUNK // LANES)
        def _(q):
            cv = cflat[pl.ds(q * LANES, LANES)]
            for l in range(LANES):
                v = cv[l]
                b0 = (v >> 4) << 4
                win = hist[pl.ds(b0, LANES)]
                hist[pl.ds(b0, LANES)] = win + jnp.where(
                    iota16 == (v - b0), 1, 0).astype(jnp.int32)

        for k in range(SGC):
            descs[k].wait()
            pltpu.sync_copy(gbuf.at[k % NBUF], acc.at[cbuf.at[k]], add=True)
            if k + NBUF < SGC:
                descs[k + NBUF] = _mk(k + NBUF)
                descs[k + NBUF].start()

    plsc.subcore_barrier()

    pltpu.sync_copy(acc.at[pl.ds(s * RPT, RPT)],
                    t_out.at[pl.ds(c * ND_PAD + s * RPT, RPT)])
    pltpu.sync_copy(hist, parts_out.at[pl.ds(w * ND_PAD, ND_PAD)])


_sc_call = pl.kernel(
    _sc_body,
    out_type=[
        jax.ShapeDtypeStruct((N_SC * ND_PAD, HALF), jnp.float32),
        jax.ShapeDtypeStruct((N_SC * N_TILE * ND_PAD,), jnp.int32),
    ],
    mesh=plsc.VectorSubcoreMesh(core_axis_name="c", subcore_axis_name="s"),
    scratch_types=[
        pltpu.VMEM((SGC, CHUNK), jnp.int32),
        pltpu.VMEM((SGC, CHUNK), jnp.int32),
        pltpu.VMEM((SGC * CHUNK,), jnp.int32),
        pltpu.VMEM((NBUF, CHUNK, HALF), jnp.float32),
        pltpu.VMEM((ND_PAD,), jnp.int32),
        pltpu.VMEM_SHARED((ND_PAD, HALF), jnp.float32),
        pltpu.SemaphoreType.DMA,
        pltpu.SemaphoreType.DMA,
    ],
)


BLK = 256


def _tc_body(h_ref, tl_ref, tr_ref, parts_ref, ws_t_ref, b_ref, w_t_ref,
             rel_ref, wrel_t_ref, o_ref):
    deg = jnp.sum(parts_ref[...], axis=0).astype(jnp.float32).reshape(BLK, 1)
    pos = deg > 0.0
    dinv = jnp.where(pos, 1.0 / deg, 0.0)
    hr = jnp.dot(rel_ref[...], wrel_t_ref[...],
                 preferred_element_type=jnp.float32)
    sl = tl_ref[...] * dinv - jnp.where(pos, hr[:, :HALF], 0.0)
    sr = tr_ref[...] * dinv - jnp.where(pos, hr[:, HALF:], 0.0)
    a = jnp.dot(h_ref[...], ws_t_ref[...], preferred_element_type=jnp.float32)
    a = a + jnp.dot(sl, w_t_ref[:HALF, :], preferred_element_type=jnp.float32)
    a = a + jnp.dot(sr, w_t_ref[HALF:, :], preferred_element_type=jnp.float32)
    a = a + b_ref[...]
    o_ref[...] = jnp.where(a > 0.0, a, jnp.exp(a) - 1.0)


_tc_call = pl.pallas_call(
    _tc_body,
    grid=(ND_PAD // BLK,),
    in_specs=[
        pl.BlockSpec((BLK, D), lambda i: (i, 0)),
        pl.BlockSpec((BLK, HALF), lambda i: (i, 0)),
        pl.BlockSpec((BLK, HALF), lambda i: (i, 0)),
        pl.BlockSpec((N_TILE, BLK), lambda i: (0, i)),
        pl.BlockSpec((D, D), lambda i: (0, 0)),
        pl.BlockSpec((1, D), lambda i: (0, 0)),
        pl.BlockSpec((D, D), lambda i: (0, 0)),
        pl.BlockSpec((1, D), lambda i: (0, 0)),
        pl.BlockSpec((D, D), lambda i: (0, 0)),
    ],
    out_specs=pl.BlockSpec((BLK, D), lambda i: (i, 0)),
    out_shape=jax.ShapeDtypeStruct((ND_PAD, D), jnp.float32),
)


def _prep_edges(row, col):
    pad = E_PAD - row.shape[0]
    rowp = jnp.concatenate([row, jnp.zeros((pad,), jnp.int32)])
    colp = jnp.concatenate([col, jnp.full((pad,), N_NODES, jnp.int32)])
    rowp = rowp.reshape(N_TILE, EC, CHUNK)
    gidx = jnp.stack([rowp * 2, rowp * 2 + 1]).reshape(N_SC * N_TILE * EC, CHUNK)
    return gidx, colp.reshape(N_TILE * EC, CHUNK), colp


def kernel(h_project, h_company, edge_index_fwd, edge_index_bwd,
           W_self_p, b_self_p, W_self_c, b_self_c,
           W_out, W_in, W_rel, rel_fwd, rel_bwd):
    ei_f = edge_index_fwd.astype(jnp.int32)
    ei_b = edge_index_bwd.astype(jnp.int32)
    gidxf, colf_t, colf_flat = _prep_edges(ei_f[0], ei_f[1])
    gidxb, colb_t, colb_flat = _prep_edges(ei_b[0], ei_b[1])
    xr_p = h_project.reshape(N_NODES * 2, HALF)
    xr_c = h_company.reshape(N_NODES * 2, HALF)
    t_c, parts_c = _sc_call(gidxf, colf_t, colf_flat, xr_p)
    xr_c, t_c = lax.optimization_barrier((xr_c, t_c))
    t_p, parts_p = _sc_call(gidxb, colb_t, colb_flat, xr_c)

    hp_pad = jnp.pad(h_project, ((0, ND_PAD - N_NODES), (0, 0)))
    hc_pad = jnp.pad(h_company, ((0, ND_PAD - N_NODES), (0, 0)))
    wrel_t = W_rel.T

    parts_p = parts_p.reshape(N_SC, N_TILE, ND_PAD)[0]
    parts_c = parts_c.reshape(N_SC, N_TILE, ND_PAD)[0]
    out_p = _tc_call(hp_pad, t_p[:ND_PAD], t_p[ND_PAD:], parts_p,
                     W_self_p.T, b_self_p.reshape(1, D), W_in.T,
                     rel_bwd.reshape(1, D), wrel_t)
    out_c = _tc_call(hc_pad, t_c[:ND_PAD], t_c[ND_PAD:], parts_c,
                     W_self_c.T, b_self_c.reshape(1, D), W_out.T,
                     rel_fwd.reshape(1, D), wrel_t)
    return out_p[:N_NODES], out_c[:N_NODES]
